# Initial kernel scaffold; baseline (speedup 1.0000x reference)
import jax
import jax.numpy as jnp
from jax.experimental import pallas as pl


def kernel(features, adj, Ws, bs):
    raise NotImplementedError("write your pallas kernel here")


# speedup vs baseline: 1.0031x; 1.0031x over previous
import jax
import jax.numpy as jnp
from jax.experimental import pallas as pl
from jax.experimental.pallas import tpu as pltpu


def _pick_rows(n):
    for r in (200, 100, 80, 40, 16, 8):
        if n % r == 0:
            return r
    return n


def _stack_kernel(features_ref, adj_ref, w0_ref, wr_ref, b_ref,
                  coords_ref, feats_out_ref,
                  support_s, x_s, feats_s, *, rows, n_layers, h):
    l = pl.program_id(0)
    r = pl.program_id(1)
    last = n_layers - 1

    @pl.when(r == 0)
    def _layer_start():
        @pl.when(l == 2)
        def _():
            feats_s[...] = (features_ref[:, :h] + x_s[...]) * 0.5

        @pl.when(jnp.logical_and(l > 2, jnp.logical_and(l % 2 == 0, l < last)))
        def _():
            feats_s[...] = (feats_s[...] + x_s[...]) * 0.5

        @pl.when(l == last)
        def _():
            feats_s[...] = (feats_s[...] + x_s[...]) * 0.5
            feats_out_ref[...] = feats_s[...]

        @pl.when(l == 0)
        def _():
            support_s[...] = jnp.dot(features_ref[...], w0_ref[...],
                                     preferred_element_type=jnp.float32)

        @pl.when(jnp.logical_and(l % 2 == 1, l != last))
        def _():
            support_s[...] = jnp.dot(x_s[...], wr_ref[0],
                                     preferred_element_type=jnp.float32)

        @pl.when(jnp.logical_or(jnp.logical_and(l % 2 == 0, l >= 2), l == last))
        def _():
            support_s[...] = jnp.dot(feats_s[...], wr_ref[0],
                                     preferred_element_type=jnp.float32)

    val = jnp.dot(adj_ref[...], support_s[...],
                  preferred_element_type=jnp.float32) + b_ref[0]

    @pl.when(l < last)
    def _():
        x_s[pl.ds(r * rows, rows), :] = jnp.maximum(val, 0.0)

    @pl.when(l == last)
    def _():
        coords_ref[pl.ds(r * rows, rows), :] = val


def kernel(features, adj, Ws, bs):
    n, f_in = features.shape
    h = Ws[0].shape[1]
    out_dim = Ws[-1].shape[1]
    n_layers = len(Ws)
    rows = _pick_rows(n)

    wr = jnp.stack([
        w if w.shape == (h, h) else
        jnp.zeros((h, h), jnp.float32).at[:, :w.shape[1]].set(w)
        for w in Ws[1:]
    ])
    bst = jnp.stack([
        (b if b.shape[0] == h else
         jnp.zeros((h,), jnp.float32).at[:b.shape[0]].set(b)).reshape(1, h)
        for b in bs
    ])

    grid = (n_layers, n // rows)
    coords_pad, feats = pl.pallas_call(
        lambda *refs: _stack_kernel(*refs, rows=rows, n_layers=n_layers, h=h),
        grid=grid,
        in_specs=[
            pl.BlockSpec((n, f_in), lambda l, r: (0, 0)),
            pl.BlockSpec((rows, n), lambda l, r: (r, 0)),
            pl.BlockSpec((f_in, h), lambda l, r: (0, 0)),
            pl.BlockSpec((1, h, h), lambda l, r: (jnp.maximum(l - 1, 0), 0, 0)),
            pl.BlockSpec((1, 1, h), lambda l, r: (l, 0, 0)),
        ],
        out_specs=[
            pl.BlockSpec((n, h), lambda l, r: (0, 0)),
            pl.BlockSpec((n, h), lambda l, r: (0, 0)),
        ],
        out_shape=[
            jax.ShapeDtypeStruct((n, h), jnp.float32),
            jax.ShapeDtypeStruct((n, h), jnp.float32),
        ],
        scratch_shapes=[
            pltpu.VMEM((n, h), jnp.float32),
            pltpu.VMEM((n, h), jnp.float32),
            pltpu.VMEM((n, h), jnp.float32),
        ],
    )(features, adj, Ws[0], wr, bst.reshape(n_layers, 1, h))
    return (coords_pad[:, :out_dim], feats)


# speedup vs baseline: 1.3831x; 1.3788x over previous
import jax
import jax.numpy as jnp
from jax.experimental import pallas as pl
from jax.experimental.pallas import tpu as pltpu


def _pick_rows(n):
    for r in (400, 200, 100, 80, 40, 16, 8):
        if n % r == 0:
            return r
    return n


def _stack_kernel(features_ref, adj_ref, w0_ref, wr_ref, b_ref,
                  coords_ref, feats_out_ref,
                  support_s, x_s, feats_s, *, rows, n_layers, h):
    l = pl.program_id(0)
    r = pl.program_id(1)
    last = n_layers - 1

    @pl.when(r == 0)
    def _layer_start():
        @pl.when(l == 2)
        def _():
            feats_s[...] = (features_ref[:, :h] + x_s[...]) * 0.5

        @pl.when(jnp.logical_and(l > 2, jnp.logical_and(l % 2 == 0, l < last)))
        def _():
            feats_s[...] = (feats_s[...] + x_s[...]) * 0.5

        @pl.when(l == last)
        def _():
            feats_s[...] = (feats_s[...] + x_s[...]) * 0.5
            feats_out_ref[...] = feats_s[...]

        @pl.when(l == 0)
        def _():
            support_s[...] = jnp.dot(features_ref[...], w0_ref[...],
                                     preferred_element_type=jnp.float32
                                     ).astype(jnp.bfloat16)

        @pl.when(jnp.logical_and(l % 2 == 1, l != last))
        def _():
            support_s[...] = jnp.dot(x_s[...], wr_ref[0],
                                     preferred_element_type=jnp.float32
                                     ).astype(jnp.bfloat16)

        @pl.when(jnp.logical_or(jnp.logical_and(l % 2 == 0, l >= 2), l == last))
        def _():
            support_s[...] = jnp.dot(feats_s[...], wr_ref[0],
                                     preferred_element_type=jnp.float32
                                     ).astype(jnp.bfloat16)

    val = jnp.dot(adj_ref[...], support_s[...],
                  preferred_element_type=jnp.float32) + b_ref[0]

    @pl.when(l < last)
    def _():
        x_s[pl.ds(r * rows, rows), :] = jnp.maximum(val, 0.0)

    @pl.when(l == last)
    def _():
        coords_ref[pl.ds(r * rows, rows), :] = val


def kernel(features, adj, Ws, bs):
    n, f_in = features.shape
    h = Ws[0].shape[1]
    out_dim = Ws[-1].shape[1]
    n_layers = len(Ws)
    rows = _pick_rows(n)

    wr = jnp.stack([
        w if w.shape == (h, h) else
        jnp.zeros((h, h), jnp.float32).at[:, :w.shape[1]].set(w)
        for w in Ws[1:]
    ])
    bst = jnp.stack([
        (b if b.shape[0] == h else
         jnp.zeros((h,), jnp.float32).at[:b.shape[0]].set(b)).reshape(1, h)
        for b in bs
    ])

    grid = (n_layers, n // rows)
    coords_pad, feats = pl.pallas_call(
        lambda *refs: _stack_kernel(*refs, rows=rows, n_layers=n_layers, h=h),
        grid=grid,
        in_specs=[
            pl.BlockSpec((n, f_in), lambda l, r: (0, 0)),
            pl.BlockSpec((rows, n), lambda l, r: (r, 0)),
            pl.BlockSpec((f_in, h), lambda l, r: (0, 0)),
            pl.BlockSpec((1, h, h), lambda l, r: (jnp.maximum(l - 1, 0), 0, 0)),
            pl.BlockSpec((1, 1, h), lambda l, r: (l, 0, 0)),
        ],
        out_specs=[
            pl.BlockSpec((n, h), lambda l, r: (0, 0)),
            pl.BlockSpec((n, h), lambda l, r: (0, 0)),
        ],
        out_shape=[
            jax.ShapeDtypeStruct((n, h), jnp.float32),
            jax.ShapeDtypeStruct((n, h), jnp.float32),
        ],
        scratch_shapes=[
            pltpu.VMEM((n, h), jnp.bfloat16),
            pltpu.VMEM((n, h), jnp.float32),
            pltpu.VMEM((n, h), jnp.float32),
        ],
    )(features, adj.astype(jnp.bfloat16), Ws[0], wr,
      bst.reshape(n_layers, 1, h))
    return (coords_pad[:, :out_dim], feats)


# speedup vs baseline: 1.4723x; 1.0645x over previous
import jax
import jax.numpy as jnp
from jax.experimental import pallas as pl
from jax.experimental.pallas import tpu as pltpu


def _pick_rows(n, candidates):
    for r in candidates:
        if n % r == 0:
            return r
    return n


def _cast_l0_kernel(features_ref, adj_ref, w0_ref, b0_ref,
                    adjb_ref, x0_ref, support_s):
    r = pl.program_id(0)

    @pl.when(r == 0)
    def _():
        support_s[...] = jnp.dot(features_ref[...], w0_ref[...],
                                 preferred_element_type=jnp.float32
                                 ).astype(jnp.bfloat16)

    ab = adj_ref[...].astype(jnp.bfloat16)
    adjb_ref[...] = ab
    val = jnp.dot(ab, support_s[...],
                  preferred_element_type=jnp.float32) + b0_ref[...]
    x0_ref[...] = jnp.maximum(val, 0.0)


def _stack_kernel(feat64_ref, x0_ref, adjb_ref, wr_ref, b_ref,
                  coords_ref, feats_out_ref,
                  support_s, x_s, feats_s, *, rows, n_layers, h):
    l = pl.program_id(0) + 1
    r = pl.program_id(1)
    last = n_layers - 1

    @pl.when(r == 0)
    def _layer_start():
        @pl.when(l == 2)
        def _():
            feats_s[...] = (feat64_ref[...] + x_s[...]) * 0.5

        @pl.when(jnp.logical_and(l > 2, jnp.logical_and(l % 2 == 0, l < last)))
        def _():
            feats_s[...] = (feats_s[...] + x_s[...]) * 0.5

        @pl.when(l == last)
        def _():
            feats_s[...] = (feats_s[...] + x_s[...]) * 0.5
            feats_out_ref[...] = feats_s[...]

        @pl.when(l == 1)
        def _():
            support_s[...] = jnp.dot(x0_ref[...], wr_ref[0],
                                     preferred_element_type=jnp.float32
                                     ).astype(jnp.bfloat16)

        @pl.when(jnp.logical_and(l % 2 == 1,
                                 jnp.logical_and(l != last, l > 1)))
        def _():
            support_s[...] = jnp.dot(x_s[...], wr_ref[0],
                                     preferred_element_type=jnp.float32
                                     ).astype(jnp.bfloat16)

        @pl.when(jnp.logical_or(jnp.logical_and(l % 2 == 0, l >= 2), l == last))
        def _():
            support_s[...] = jnp.dot(feats_s[...], wr_ref[0],
                                     preferred_element_type=jnp.float32
                                     ).astype(jnp.bfloat16)

    val = jnp.dot(adjb_ref[...], support_s[...],
                  preferred_element_type=jnp.float32) + b_ref[0]

    @pl.when(l < last)
    def _():
        x_s[pl.ds(r * rows, rows), :] = jnp.maximum(val, 0.0)

    @pl.when(l == last)
    def _():
        coords_ref[pl.ds(r * rows, rows), :] = val


def kernel(features, adj, Ws, bs):
    n, f_in = features.shape
    h = Ws[0].shape[1]
    out_dim = Ws[-1].shape[1]
    n_layers = len(Ws)

    wr = jnp.stack([
        w if w.shape == (h, h) else
        jnp.zeros((h, h), jnp.float32).at[:, :w.shape[1]].set(w)
        for w in Ws[1:]
    ])
    bst = jnp.stack([
        (b if b.shape[0] == h else
         jnp.zeros((h,), jnp.float32).at[:b.shape[0]].set(b)).reshape(1, h)
        for b in bs[1:]
    ])

    rows_a = _pick_rows(n, (200, 100, 80, 40, 16, 8))
    adjb, x0 = pl.pallas_call(
        _cast_l0_kernel,
        grid=(n // rows_a,),
        in_specs=[
            pl.BlockSpec((n, f_in), lambda r: (0, 0)),
            pl.BlockSpec((rows_a, n), lambda r: (r, 0)),
            pl.BlockSpec((f_in, h), lambda r: (0, 0)),
            pl.BlockSpec((1, h), lambda r: (0, 0)),
        ],
        out_specs=[
            pl.BlockSpec((rows_a, n), lambda r: (r, 0)),
            pl.BlockSpec((rows_a, h), lambda r: (r, 0)),
        ],
        out_shape=[
            jax.ShapeDtypeStruct((n, n), jnp.bfloat16),
            jax.ShapeDtypeStruct((n, h), jnp.float32),
        ],
        scratch_shapes=[pltpu.VMEM((n, h), jnp.bfloat16)],
    )(features, adj, Ws[0], bs[0].reshape(1, h))

    rows = _pick_rows(n, (400, 200, 100, 80, 40, 16, 8))
    grid = (n_layers - 1, n // rows)
    coords_pad, feats = pl.pallas_call(
        lambda *refs: _stack_kernel(*refs, rows=rows, n_layers=n_layers, h=h),
        grid=grid,
        in_specs=[
            pl.BlockSpec((n, h), lambda l, r: (0, 0)),
            pl.BlockSpec((n, h), lambda l, r: (0, 0)),
            pl.BlockSpec((rows, n), lambda l, r: (r, 0)),
            pl.BlockSpec((1, h, h), lambda l, r: (l, 0, 0)),
            pl.BlockSpec((1, 1, h), lambda l, r: (l, 0, 0)),
        ],
        out_specs=[
            pl.BlockSpec((n, h), lambda l, r: (0, 0)),
            pl.BlockSpec((n, h), lambda l, r: (0, 0)),
        ],
        out_shape=[
            jax.ShapeDtypeStruct((n, h), jnp.float32),
            jax.ShapeDtypeStruct((n, h), jnp.float32),
        ],
        scratch_shapes=[
            pltpu.VMEM((n, h), jnp.bfloat16),
            pltpu.VMEM((n, h), jnp.float32),
            pltpu.VMEM((n, h), jnp.float32),
        ],
    )(features[:, :h], x0, adjb, wr, bst.reshape(n_layers - 1, 1, h))
    return (coords_pad[:, :out_dim], feats)
